# baseline (speedup 1.0000x reference)
import functools

import jax
import jax.numpy as jnp
from jax import lax
from jax.experimental import pallas as pl
from jax.experimental.pallas import tpu as pltpu
from jax.experimental.pallas import tpu_sc as plsc

N = 10000
D = 128
E = 320000

NC = 2
NS = 16
NW = NC * NS
EPT = E // NW
K = 100
SB = 50
NB = EPT // (SB * K)
NPAD = 10240
DEG_RPT = NPAD // NS
ACC_RPT = NPAD // NS
ZR = 64

_MESH = plsc.VectorSubcoreMesh(core_axis_name="c", subcore_axis_name="s",
                               num_cores=NC, num_subcores=NS)


DEG_UNROLL = 25


def _sc_degrees(src_hbm, dst_hbm, z1_hbm, dega_hbm, degb_hbm,
                src_v, dst_v, hist_a, hist_b):
  c = lax.axis_index("c")
  s = lax.axis_index("s")
  w = c * NS + s

  pltpu.sync_copy(z1_hbm, hist_a)
  pltpu.sync_copy(z1_hbm, hist_b)
  pltpu.sync_copy(src_hbm.at[pl.ds(w * EPT, EPT)], src_v)
  pltpu.sync_copy(dst_hbm.at[pl.ds(w * EPT, EPT)], dst_v)

  ones16 = jnp.full((16,), 1.0, jnp.float32)

  @pl.loop(0, EPT // 16, step=DEG_UNROLL)
  def _(j):
    for i in range(DEG_UNROLL):
      nida = src_v[pl.ds((j + i) * 16, 16)]
      plsc.addupdate_scatter(hist_a, [nida], ones16)
      nidb = dst_v[pl.ds((j + i) * 16, 16)]
      plsc.addupdate_scatter(hist_b, [nidb], ones16)

  pltpu.sync_copy(hist_a, dega_hbm.at[pl.ds(w * NPAD, NPAD)])
  pltpu.sync_copy(hist_b, degb_hbm.at[pl.ds(w * NPAD, NPAD)])


_sc_degrees_call = functools.partial(
    pl.kernel,
    out_type=[jax.ShapeDtypeStruct((NW * NPAD,), jnp.float32),
              jax.ShapeDtypeStruct((NW * NPAD,), jnp.float32)],
    mesh=_MESH,
    scratch_types=[
        pltpu.VMEM((EPT,), jnp.int32),
        pltpu.VMEM((EPT,), jnp.int32),
        pltpu.VMEM((NPAD,), jnp.float32),
        pltpu.VMEM((NPAD,), jnp.float32),
    ],
    compiler_params=pltpu.CompilerParams(needs_layout_passes=False),
)(_sc_degrees)


def _sc_scatter(scaled_hbm, src_hbm, dst_hbm, z2_hbm, part_hbm,
                src_v, dst_v, rows0, rows1,
                gs0, gs1, ss0, ss1, acc_sp):
  c = lax.axis_index("c")
  s = lax.axis_index("s")
  w = c * NS + s
  rows = [rows0, rows1]
  gsem = [gs0, gs1]
  ssem = [ss0, ss1]

  zd = [pltpu.async_copy(z2_hbm,
                         acc_sp.at[pl.ds(s * ACC_RPT + i * ZR, ZR), :],
                         gs0)
        for i in range(ACC_RPT // ZR)]
  for d in zd:
    d.wait()

  plsc.subcore_barrier()

  @pl.loop(0, NB)
  def _(t):
    pltpu.sync_copy(src_hbm.at[w, t], src_v)
    pltpu.sync_copy(dst_hbm.at[w, t], dst_v)
    gd = {}
    sd = {}

    def scat(r):
      gd[r].wait()
      b = r % 2
      sd[r] = pltpu.async_copy(rows[b], acc_sp.at[dst_v.at[r]], ssem[b],
                               add=True)

    for r in range(SB):
      b = r % 2
      if r >= 2:
        sd[r - 2].wait()
      gd[r] = pltpu.async_copy(scaled_hbm.at[src_v.at[r]], rows[b], gsem[b])
      if r >= 1:
        scat(r - 1)
    scat(SB - 1)
    for r in range(SB - 2, SB):
      sd[r].wait()

  plsc.subcore_barrier()

  sl = pl.ds(s * ACC_RPT, ACC_RPT)
  pltpu.sync_copy(acc_sp.at[sl, :], part_hbm.at[c, sl, :])


_sc_scatter_call = functools.partial(
    pl.kernel,
    out_type=jax.ShapeDtypeStruct((NC, NPAD, D), jnp.float32),
    mesh=_MESH,
    scratch_types=[
        pltpu.VMEM((SB, K), jnp.int32),
        pltpu.VMEM((SB, K), jnp.int32),
        pltpu.VMEM((K, D), jnp.float32),
        pltpu.VMEM((K, D), jnp.float32),
        pltpu.SemaphoreType.DMA,
        pltpu.SemaphoreType.DMA,
        pltpu.SemaphoreType.DMA,
        pltpu.SemaphoreType.DMA,
        pltpu.VMEM_SHARED((NPAD, D), jnp.float32),
    ],
)(_sc_scatter)


def _tc_prep(dega_ref, degb_ref, feats_ref, scaled_ref, norm_in_ref):
  deg_out = jnp.sum(dega_ref[...], axis=0)
  deg_in = jnp.sum(degb_ref[...], axis=0)
  norm_out = jnp.where(deg_out > 0.0,
                       lax.rsqrt(jnp.maximum(deg_out, 1.0)), 0.0)
  norm_in = jnp.where(deg_in > 0.0,
                      lax.rsqrt(jnp.maximum(deg_in, 1.0)), 0.0)
  scaled_ref[...] = feats_ref[...] * norm_out[:N][:, None]
  norm_in_ref[...] = norm_in[:N][:, None]


def _tc_prep_call(dega, degb, feats):
  return pl.pallas_call(
      _tc_prep,
      out_shape=[jax.ShapeDtypeStruct((N, D), jnp.float32),
                 jax.ShapeDtypeStruct((N, 1), jnp.float32)],
      compiler_params=pltpu.CompilerParams(
          vmem_limit_bytes=60 * 1024 * 1024),
  )(dega, degb, feats)


BR = 2000


def _tc_dense(part_ref, norm_ref, feats_ref, w_ref, wr_ref, bb_ref,
              gamma_ref, beta_ref, out_ref, y_scr, stat_scr):
  p = pl.program_id(0)
  j = pl.program_id(1)

  @pl.when(p == 0)
  def _():
    agg = (part_ref[0] + part_ref[1]) * norm_ref[...]
    y = (jnp.dot(agg, w_ref[...], preferred_element_type=jnp.float32,
                 precision=lax.Precision.HIGHEST)
         + jnp.dot(feats_ref[...], wr_ref[...],
                   preferred_element_type=jnp.float32,
                   precision=lax.Precision.HIGHEST)
         + bb_ref[...])
    y_scr[pl.ds(j * BR, BR), :] = y
    blk = jnp.concatenate([jnp.sum(y, axis=0, keepdims=True),
                           jnp.sum(y * y, axis=0, keepdims=True)], axis=0)

    @pl.when(j == 0)
    def _():
      stat_scr[...] = blk

    @pl.when(j > 0)
    def _():
      stat_scr[...] += blk

  @pl.when(p == 1)
  def _():
    mean = stat_scr[0:1, :] * (1.0 / N)
    var = stat_scr[1:2, :] * (1.0 / N) - mean * mean
    y = y_scr[pl.ds(j * BR, BR), :]
    out_ref[...] = ((y - mean) * lax.rsqrt(var + 1e-5) * gamma_ref[...]
                    + beta_ref[...])


def _tc_dense_call(part, norm_in, feats, W, W_res, bb, gamma, beta):
  first = lambda p, j: (0, jnp.where(p == 0, j, 0), 0)
  first2 = lambda p, j: (jnp.where(p == 0, j, 0), 0)
  whole = lambda p, j: (0, 0)
  return pl.pallas_call(
      _tc_dense,
      grid=(2, N // BR),
      in_specs=[
          pl.BlockSpec((NC, BR, D), first),
          pl.BlockSpec((BR, 1), first2),
          pl.BlockSpec((BR, D), first2),
          pl.BlockSpec((D, D), whole),
          pl.BlockSpec((D, D), whole),
          pl.BlockSpec((1, D), whole),
          pl.BlockSpec((1, D), whole),
          pl.BlockSpec((1, D), whole),
      ],
      out_specs=pl.BlockSpec((BR, D), lambda p, j: (j, 0)),
      out_shape=jax.ShapeDtypeStruct((N, D), jnp.float32),
      scratch_shapes=[pltpu.VMEM((N, D), jnp.float32),
                      pltpu.VMEM((2, D), jnp.float32)],
      compiler_params=pltpu.CompilerParams(
          vmem_limit_bytes=58 * 1024 * 1024,
          dimension_semantics=("arbitrary", "arbitrary")),
  )(part, norm_in, feats, W, W_res, bb, gamma, beta)


def kernel(feats, edge_index, W, b, W_res, b_res, gamma, beta):
  src4 = edge_index[0].reshape(NW, NB, SB, K)
  dst4 = edge_index[1].reshape(NW, NB, SB, K)
  z1 = jnp.zeros((NPAD,), jnp.float32)
  z2 = jnp.zeros((ZR, D), jnp.float32)

  dega, degb = _sc_degrees_call(edge_index[0], edge_index[1], z1)
  scaled, norm_in = _tc_prep_call(dega.reshape(NW, NPAD),
                                  degb.reshape(NW, NPAD), feats)
  part = _sc_scatter_call(scaled, src4, dst4, z2)

  bb = (b + b_res).reshape(1, D)
  return _tc_dense_call(part, norm_in, feats, W, W_res, bb,
                        gamma.reshape(1, D), beta.reshape(1, D))


# speedup vs baseline: 1.0436x; 1.0436x over previous
import functools

import jax
import jax.numpy as jnp
from jax import lax
from jax.experimental import pallas as pl
from jax.experimental.pallas import tpu as pltpu
from jax.experimental.pallas import tpu_sc as plsc

N = 10000
D = 128
E = 320000

NC = 2
NS = 16
NW = NC * NS
EPT = E // NW
K = 100
SB = 25
NB = EPT // (SB * K)
NPAD = 10240
DEG_RPT = NPAD // NS
ACC_RPT = NPAD // NS
ZR = 64

_MESH = plsc.VectorSubcoreMesh(core_axis_name="c", subcore_axis_name="s",
                               num_cores=NC, num_subcores=NS)


DEG_UNROLL = 25


def _sc_degrees(src_hbm, dst_hbm, z1_hbm, dega_hbm, degb_hbm,
                src_v, dst_v, hist_a, hist_b):
  c = lax.axis_index("c")
  s = lax.axis_index("s")
  w = c * NS + s

  pltpu.sync_copy(z1_hbm, hist_a)
  pltpu.sync_copy(z1_hbm, hist_b)
  pltpu.sync_copy(src_hbm.at[pl.ds(w * EPT, EPT)], src_v)
  pltpu.sync_copy(dst_hbm.at[pl.ds(w * EPT, EPT)], dst_v)

  ones16 = jnp.full((16,), 1.0, jnp.float32)

  @pl.loop(0, EPT // 16, step=DEG_UNROLL)
  def _(j):
    for i in range(DEG_UNROLL):
      nida = src_v[pl.ds((j + i) * 16, 16)]
      plsc.addupdate_scatter(hist_a, [nida], ones16)
      nidb = dst_v[pl.ds((j + i) * 16, 16)]
      plsc.addupdate_scatter(hist_b, [nidb], ones16)

  pltpu.sync_copy(hist_a, dega_hbm.at[pl.ds(w * NPAD, NPAD)])
  pltpu.sync_copy(hist_b, degb_hbm.at[pl.ds(w * NPAD, NPAD)])


_sc_degrees_call = functools.partial(
    pl.kernel,
    out_type=[jax.ShapeDtypeStruct((NW * NPAD,), jnp.float32),
              jax.ShapeDtypeStruct((NW * NPAD,), jnp.float32)],
    mesh=_MESH,
    scratch_types=[
        pltpu.VMEM((EPT,), jnp.int32),
        pltpu.VMEM((EPT,), jnp.int32),
        pltpu.VMEM((NPAD,), jnp.float32),
        pltpu.VMEM((NPAD,), jnp.float32),
    ],
    compiler_params=pltpu.CompilerParams(needs_layout_passes=False),
)(_sc_degrees)


def _sc_scatter(scaled_hbm, src_hbm, dst_hbm, z2_hbm, part_hbm,
                src_v, dst_v, rows0, rows1, rows2,
                gs0, gs1, gs2, ss0, ss1, ss2, acc_sp):
  c = lax.axis_index("c")
  s = lax.axis_index("s")
  w = c * NS + s
  rows = [rows0, rows1, rows2]
  gsem = [gs0, gs1, gs2]
  ssem = [ss0, ss1, ss2]

  zd = [pltpu.async_copy(z2_hbm,
                         acc_sp.at[pl.ds(s * ACC_RPT + i * ZR, ZR), :],
                         gs0)
        for i in range(ACC_RPT // ZR)]
  for d in zd:
    d.wait()

  plsc.subcore_barrier()

  @pl.loop(0, NB)
  def _(t):
    pltpu.sync_copy(src_hbm.at[w, t], src_v)
    pltpu.sync_copy(dst_hbm.at[w, t], dst_v)
    gd = {}
    sd = {}

    def scat(r):
      gd[r].wait()
      b = r % 3
      sd[r] = pltpu.async_copy(rows[b], acc_sp.at[dst_v.at[r]], ssem[b],
                               add=True)

    for r in range(SB):
      b = r % 3
      if r >= 3:
        sd[r - 3].wait()
      gd[r] = pltpu.async_copy(scaled_hbm.at[src_v.at[r]], rows[b], gsem[b])
      if r >= 2:
        scat(r - 2)
    scat(SB - 2)
    scat(SB - 1)
    for r in range(SB - 3, SB):
      sd[r].wait()

  plsc.subcore_barrier()

  sl = pl.ds(s * ACC_RPT, ACC_RPT)
  pltpu.sync_copy(acc_sp.at[sl, :], part_hbm.at[c, sl, :])


_sc_scatter_call = functools.partial(
    pl.kernel,
    out_type=jax.ShapeDtypeStruct((NC, NPAD, D), jnp.float32),
    mesh=_MESH,
    scratch_types=[
        pltpu.VMEM((SB, K), jnp.int32),
        pltpu.VMEM((SB, K), jnp.int32),
        pltpu.VMEM((K, D), jnp.float32),
        pltpu.VMEM((K, D), jnp.float32),
        pltpu.VMEM((K, D), jnp.float32),
        pltpu.SemaphoreType.DMA,
        pltpu.SemaphoreType.DMA,
        pltpu.SemaphoreType.DMA,
        pltpu.SemaphoreType.DMA,
        pltpu.SemaphoreType.DMA,
        pltpu.SemaphoreType.DMA,
        pltpu.VMEM_SHARED((NPAD, D), jnp.float32),
    ],
)(_sc_scatter)


def _tc_prep(dega_ref, degb_ref, feats_ref, scaled_ref, norm_in_ref):
  deg_out = jnp.sum(dega_ref[...], axis=0)
  deg_in = jnp.sum(degb_ref[...], axis=0)
  norm_out = jnp.where(deg_out > 0.0,
                       lax.rsqrt(jnp.maximum(deg_out, 1.0)), 0.0)
  norm_in = jnp.where(deg_in > 0.0,
                      lax.rsqrt(jnp.maximum(deg_in, 1.0)), 0.0)
  scaled_ref[...] = feats_ref[...] * norm_out[:N][:, None]
  norm_in_ref[...] = norm_in[:N][:, None]


def _tc_prep_call(dega, degb, feats):
  return pl.pallas_call(
      _tc_prep,
      out_shape=[jax.ShapeDtypeStruct((N, D), jnp.float32),
                 jax.ShapeDtypeStruct((N, 1), jnp.float32)],
      compiler_params=pltpu.CompilerParams(
          vmem_limit_bytes=60 * 1024 * 1024),
  )(dega, degb, feats)


BR = 2000


def _tc_dense(part_ref, norm_ref, feats_ref, w_ref, wr_ref, bb_ref,
              gamma_ref, beta_ref, out_ref, y_scr, stat_scr):
  p = pl.program_id(0)
  j = pl.program_id(1)

  @pl.when(p == 0)
  def _():
    agg = (part_ref[0] + part_ref[1]) * norm_ref[...]
    y = (jnp.dot(agg, w_ref[...], preferred_element_type=jnp.float32,
                 precision=lax.Precision.HIGHEST)
         + jnp.dot(feats_ref[...], wr_ref[...],
                   preferred_element_type=jnp.float32,
                   precision=lax.Precision.HIGHEST)
         + bb_ref[...])
    y_scr[pl.ds(j * BR, BR), :] = y
    blk = jnp.concatenate([jnp.sum(y, axis=0, keepdims=True),
                           jnp.sum(y * y, axis=0, keepdims=True)], axis=0)

    @pl.when(j == 0)
    def _():
      stat_scr[...] = blk

    @pl.when(j > 0)
    def _():
      stat_scr[...] += blk

  @pl.when(p == 1)
  def _():
    mean = stat_scr[0:1, :] * (1.0 / N)
    var = stat_scr[1:2, :] * (1.0 / N) - mean * mean
    y = y_scr[pl.ds(j * BR, BR), :]
    out_ref[...] = ((y - mean) * lax.rsqrt(var + 1e-5) * gamma_ref[...]
                    + beta_ref[...])


def _tc_dense_call(part, norm_in, feats, W, W_res, bb, gamma, beta):
  first = lambda p, j: (0, jnp.where(p == 0, j, 0), 0)
  first2 = lambda p, j: (jnp.where(p == 0, j, 0), 0)
  whole = lambda p, j: (0, 0)
  return pl.pallas_call(
      _tc_dense,
      grid=(2, N // BR),
      in_specs=[
          pl.BlockSpec((NC, BR, D), first),
          pl.BlockSpec((BR, 1), first2),
          pl.BlockSpec((BR, D), first2),
          pl.BlockSpec((D, D), whole),
          pl.BlockSpec((D, D), whole),
          pl.BlockSpec((1, D), whole),
          pl.BlockSpec((1, D), whole),
          pl.BlockSpec((1, D), whole),
      ],
      out_specs=pl.BlockSpec((BR, D), lambda p, j: (j, 0)),
      out_shape=jax.ShapeDtypeStruct((N, D), jnp.float32),
      scratch_shapes=[pltpu.VMEM((N, D), jnp.float32),
                      pltpu.VMEM((2, D), jnp.float32)],
      compiler_params=pltpu.CompilerParams(
          vmem_limit_bytes=58 * 1024 * 1024,
          dimension_semantics=("arbitrary", "arbitrary")),
  )(part, norm_in, feats, W, W_res, bb, gamma, beta)


def kernel(feats, edge_index, W, b, W_res, b_res, gamma, beta):
  src4 = edge_index[0].reshape(NW, NB, SB, K)
  dst4 = edge_index[1].reshape(NW, NB, SB, K)
  z1 = jnp.zeros((NPAD,), jnp.float32)
  z2 = jnp.zeros((ZR, D), jnp.float32)

  dega, degb = _sc_degrees_call(edge_index[0], edge_index[1], z1)
  scaled, norm_in = _tc_prep_call(dega.reshape(NW, NPAD),
                                  degb.reshape(NW, NPAD), feats)
  part = _sc_scatter_call(scaled, src4, dst4, z2)

  bb = (b + b_res).reshape(1, D)
  return _tc_dense_call(part, norm_in, feats, W, W_res, bb,
                        gamma.reshape(1, D), beta.reshape(1, D))
